# baseline (speedup 1.0000x reference)
import jax
import jax.numpy as jnp
from jax.experimental import pallas as pl
from jax.experimental.pallas import tpu as pltpu

_TM = 1024
_TF = 1024


def _ffn_kernel(x_ref, c0_ref, g_ref, b_ref, w1_ref, b1_ref, w2_ref, b2_ref,
                o_ref, normed_ref, alpha_ref, h_ref):
    f = pl.program_id(1)
    nf = pl.num_programs(1) - 1

    @pl.when(f == 0)
    def _init():
        x = x_ref[...]
        mu = jnp.mean(x, axis=1, keepdims=True)
        var = jnp.mean(x * x, axis=1, keepdims=True) - mu * mu
        normed = (x - mu) * jax.lax.rsqrt(var + 1e-5) * g_ref[...] + b_ref[...]
        normed_ref[...] = normed.astype(jnp.float8_e4m3fn)
        alpha = jax.nn.sigmoid(jax.lax.dot_general(
            x, c0_ref[...], (((1,), (1,)), ((), ())),
            preferred_element_type=jnp.float32))
        alpha_ref[...] = alpha * 0.0625
        o_ref[...] = x + alpha * b2_ref[...]

    @pl.when(f > 0)
    def _dot2():
        o_ref[...] += jax.lax.dot_general(
            h_ref[(f - 1) % 2], w2_ref[...], (((1,), (1,)), ((), ())),
            preferred_element_type=jnp.float32)

    @pl.when(f < nf)
    def _dot1():
        h = jax.lax.dot_general(
            normed_ref[...], w1_ref[...], (((1,), (1,)), ((), ())),
            preferred_element_type=jnp.float32)
        h_ref[f % 2] = (jnp.maximum(h + b1_ref[...], 0.0)
                        * alpha_ref[...]).astype(jnp.float8_e4m3fn)


def kernel(input_features, expert_centroids, ln_g, ln_b, ff1_w, ff1_b,
           ff2_w, ff2_b):
    orig_shape = input_features.shape
    d = orig_shape[-1]
    x = input_features.reshape(-1, d)
    n = x.shape[0]
    ff = ff1_w.shape[0]
    nf = ff // _TF

    c0 = expert_centroids[0:1]
    g = ln_g.reshape(1, d)
    b = ln_b.reshape(1, d)
    w1 = (ff1_w * 16.0).astype(jnp.float8_e4m3fn)
    b1 = (ff1_b * 16.0).reshape(1, ff)
    w2 = ff2_w.astype(jnp.float8_e4m3fn)
    b2 = ff2_b.reshape(1, d)

    grid = (n // _TM, nf + 1)
    out = pl.pallas_call(
        _ffn_kernel,
        grid=grid,
        in_specs=[
            pl.BlockSpec((_TM, d), lambda m, f: (m, 0)),
            pl.BlockSpec((1, d), lambda m, f: (0, 0)),
            pl.BlockSpec((1, d), lambda m, f: (0, 0)),
            pl.BlockSpec((1, d), lambda m, f: (0, 0)),
            pl.BlockSpec((_TF, d),
                         lambda m, f: (jnp.minimum(f, nf - 1), 0)),
            pl.BlockSpec((1, _TF),
                         lambda m, f: (0, jnp.minimum(f, nf - 1))),
            pl.BlockSpec((d, _TF),
                         lambda m, f: (0, jnp.maximum(f - 1, 0))),
            pl.BlockSpec((1, d), lambda m, f: (0, 0)),
        ],
        out_specs=pl.BlockSpec((_TM, d), lambda m, f: (m, 0)),
        out_shape=jax.ShapeDtypeStruct((n, d), jnp.float32),
        scratch_shapes=[
            pltpu.VMEM((_TM, d), jnp.float8_e4m3fn),
            pltpu.VMEM((_TM, 1), jnp.float32),
            pltpu.VMEM((2, _TM, _TF), jnp.float8_e4m3fn),
        ],
        compiler_params=pltpu.CompilerParams(
            dimension_semantics=("parallel", "arbitrary")),
    )(x, c0, g, b, w1, b1, w2, b2)
    return out.reshape(orig_shape)


# speedup vs baseline: 1.1078x; 1.1078x over previous
import jax
import jax.numpy as jnp
from jax.experimental import pallas as pl
from jax.experimental.pallas import tpu as pltpu

_TM = 1024
_TF = 1024


def _ffn_kernel(x_ref, c0_ref, g_ref, b_ref, w1_ref, b1_ref, w2_ref, b2_ref,
                o_ref, normed_ref, alpha_ref):
    f = pl.program_id(1)

    @pl.when(f == 0)
    def _init():
        x = x_ref[...]
        mu = jnp.mean(x, axis=1, keepdims=True)
        var = jnp.mean(x * x, axis=1, keepdims=True) - mu * mu
        normed = (x - mu) * jax.lax.rsqrt(var + 1e-5) * g_ref[...] + b_ref[...]
        normed_ref[...] = normed.astype(jnp.float8_e4m3fn)
        alpha = jax.nn.sigmoid(jax.lax.dot_general(
            x, c0_ref[...], (((1,), (1,)), ((), ())),
            preferred_element_type=jnp.float32))
        alpha_ref[...] = alpha * 0.0625
        o_ref[...] = x + alpha * b2_ref[...]

    h = jax.lax.dot_general(
        normed_ref[...], w1_ref[...], (((1,), (1,)), ((), ())),
        preferred_element_type=jnp.float32)
    h = (jnp.maximum(h + b1_ref[...], 0.0)
         * alpha_ref[...]).astype(jnp.float8_e4m3fn)
    o_ref[...] += jax.lax.dot_general(
        h, w2_ref[...], (((1,), (1,)), ((), ())),
        preferred_element_type=jnp.float32)


def kernel(input_features, expert_centroids, ln_g, ln_b, ff1_w, ff1_b,
           ff2_w, ff2_b):
    orig_shape = input_features.shape
    d = orig_shape[-1]
    x = input_features.reshape(-1, d)
    n = x.shape[0]
    ff = ff1_w.shape[0]

    c0 = expert_centroids[0:1]
    g = ln_g.reshape(1, d)
    b = ln_b.reshape(1, d)
    w1 = (ff1_w * 16.0).astype(jnp.float8_e4m3fn)
    b1 = (ff1_b * 16.0).reshape(1, ff)
    w2 = ff2_w.astype(jnp.float8_e4m3fn)
    b2 = ff2_b.reshape(1, d)

    grid = (n // _TM, ff // _TF)
    out = pl.pallas_call(
        _ffn_kernel,
        grid=grid,
        in_specs=[
            pl.BlockSpec((_TM, d), lambda m, f: (m, 0)),
            pl.BlockSpec((1, d), lambda m, f: (0, 0)),
            pl.BlockSpec((1, d), lambda m, f: (0, 0)),
            pl.BlockSpec((1, d), lambda m, f: (0, 0)),
            pl.BlockSpec((_TF, d), lambda m, f: (f, 0)),
            pl.BlockSpec((1, _TF), lambda m, f: (0, f)),
            pl.BlockSpec((d, _TF), lambda m, f: (0, f)),
            pl.BlockSpec((1, d), lambda m, f: (0, 0)),
        ],
        out_specs=pl.BlockSpec((_TM, d), lambda m, f: (m, 0)),
        out_shape=jax.ShapeDtypeStruct((n, d), jnp.float32),
        scratch_shapes=[
            pltpu.VMEM((_TM, d), jnp.float8_e4m3fn),
            pltpu.VMEM((_TM, 1), jnp.float32),
        ],
        compiler_params=pltpu.CompilerParams(
            dimension_semantics=("parallel", "arbitrary")),
    )(x, c0, g, b, w1, b1, w2, b2)
    return out.reshape(orig_shape)


# speedup vs baseline: 1.1534x; 1.0412x over previous
import jax
import jax.numpy as jnp
from jax.experimental import pallas as pl
from jax.experimental.pallas import tpu as pltpu

_TM = 1024
_TF = 1024


def _ffn_kernel(x_ref, c0_ref, g_ref, b_ref, w1_ref, b1_ref, w2_ref, b2_ref,
                o_ref, normed_ref, alpha_ref):
    f = pl.program_id(1)

    def _dot_pair(normed_fp8, gate):
        h = jax.lax.dot_general(
            normed_fp8, w1_ref[...], (((1,), (1,)), ((), ())),
            preferred_element_type=jnp.float32)
        h = (jnp.maximum(h + b1_ref[...], 0.0)
             * gate).astype(jnp.float8_e4m3fn)
        o_ref[...] += jax.lax.dot_general(
            h, w2_ref[...], (((1,), (1,)), ((), ())),
            preferred_element_type=jnp.float32)

    @pl.when(f == 0)
    def _init():
        x = x_ref[...]
        mu = jnp.mean(x, axis=1, keepdims=True)
        var = jnp.mean(x * x, axis=1, keepdims=True) - mu * mu
        normed = (x - mu) * jax.lax.rsqrt(var + 1e-5) * g_ref[...] + b_ref[...]
        normed_fp8 = normed.astype(jnp.float8_e4m3fn)
        normed_ref[...] = normed_fp8
        alpha = jax.nn.sigmoid(jax.lax.dot_general(
            x, c0_ref[...], (((1,), (1,)), ((), ())),
            preferred_element_type=jnp.float32))
        gate = alpha * 0.0625
        alpha_ref[...] = gate
        o_ref[...] = x + alpha * b2_ref[...]
        _dot_pair(normed_fp8, gate)

    @pl.when(f > 0)
    def _steady():
        _dot_pair(normed_ref[...], alpha_ref[...])


def kernel(input_features, expert_centroids, ln_g, ln_b, ff1_w, ff1_b,
           ff2_w, ff2_b):
    orig_shape = input_features.shape
    d = orig_shape[-1]
    x = input_features.reshape(-1, d)
    n = x.shape[0]
    ff = ff1_w.shape[0]

    c0 = expert_centroids[0:1]
    g = ln_g.reshape(1, d)
    b = ln_b.reshape(1, d)
    w1 = (ff1_w * 16.0).astype(jnp.float8_e4m3fn)
    b1 = (ff1_b * 16.0).reshape(1, ff)
    w2 = ff2_w.astype(jnp.float8_e4m3fn)
    b2 = ff2_b.reshape(1, d)

    grid = (n // _TM, ff // _TF)
    out = pl.pallas_call(
        _ffn_kernel,
        grid=grid,
        in_specs=[
            pl.BlockSpec((_TM, d), lambda m, f: (m, 0)),
            pl.BlockSpec((1, d), lambda m, f: (0, 0)),
            pl.BlockSpec((1, d), lambda m, f: (0, 0)),
            pl.BlockSpec((1, d), lambda m, f: (0, 0)),
            pl.BlockSpec((_TF, d), lambda m, f: (f, 0)),
            pl.BlockSpec((1, _TF), lambda m, f: (0, f)),
            pl.BlockSpec((d, _TF), lambda m, f: (0, f)),
            pl.BlockSpec((1, d), lambda m, f: (0, 0)),
        ],
        out_specs=pl.BlockSpec((_TM, d), lambda m, f: (m, 0)),
        out_shape=jax.ShapeDtypeStruct((n, d), jnp.float32),
        scratch_shapes=[
            pltpu.VMEM((_TM, d), jnp.float8_e4m3fn),
            pltpu.VMEM((_TM, 1), jnp.float32),
        ],
        compiler_params=pltpu.CompilerParams(
            dimension_semantics=("parallel", "arbitrary")),
    )(x, c0, g, b, w1, b1, w2, b2)
    return out.reshape(orig_shape)


# speedup vs baseline: 1.1727x; 1.0167x over previous
import jax
import jax.numpy as jnp
from jax.experimental import pallas as pl
from jax.experimental.pallas import tpu as pltpu

_TM = 1024
_TF = 2048
_TH = 1024


def _ffn_kernel(x_ref, c0_ref, g_ref, b_ref, w1_ref, b1_ref, w2_ref, b2_ref,
                o_ref, normed_ref, alpha_ref):
    f = pl.program_id(1)

    def _dot_pair(normed_fp8, gate):
        for k in range(_TF // _TH):
            sl = slice(k * _TH, (k + 1) * _TH)
            h = jax.lax.dot_general(
                normed_fp8, w1_ref[sl, :], (((1,), (1,)), ((), ())),
                preferred_element_type=jnp.float32)
            h = (jnp.maximum(h + b1_ref[:, sl], 0.0)
                 * gate).astype(jnp.float8_e4m3fn)
            o_ref[...] += jax.lax.dot_general(
                h, w2_ref[:, sl], (((1,), (1,)), ((), ())),
                preferred_element_type=jnp.float32)

    @pl.when(f == 0)
    def _init():
        x = x_ref[...]
        mu = jnp.mean(x, axis=1, keepdims=True)
        var = jnp.mean(x * x, axis=1, keepdims=True) - mu * mu
        normed = (x - mu) * jax.lax.rsqrt(var + 1e-5) * g_ref[...] + b_ref[...]
        normed_fp8 = normed.astype(jnp.float8_e4m3fn)
        normed_ref[...] = normed_fp8
        alpha = jax.nn.sigmoid(jax.lax.dot_general(
            x, c0_ref[...], (((1,), (1,)), ((), ())),
            preferred_element_type=jnp.float32))
        gate = alpha * 0.0625
        alpha_ref[...] = gate
        o_ref[...] = x + alpha * b2_ref[...]
        _dot_pair(normed_fp8, gate)

    @pl.when(f > 0)
    def _steady():
        _dot_pair(normed_ref[...], alpha_ref[...])


def kernel(input_features, expert_centroids, ln_g, ln_b, ff1_w, ff1_b,
           ff2_w, ff2_b):
    orig_shape = input_features.shape
    d = orig_shape[-1]
    x = input_features.reshape(-1, d)
    n = x.shape[0]
    ff = ff1_w.shape[0]

    c0 = expert_centroids[0:1]
    g = ln_g.reshape(1, d)
    b = ln_b.reshape(1, d)
    w1 = (ff1_w * 16.0).astype(jnp.float8_e4m3fn)
    b1 = (ff1_b * 16.0).reshape(1, ff)
    w2 = ff2_w.astype(jnp.float8_e4m3fn)
    b2 = ff2_b.reshape(1, d)

    grid = (n // _TM, ff // _TF)
    out = pl.pallas_call(
        _ffn_kernel,
        grid=grid,
        in_specs=[
            pl.BlockSpec((_TM, d), lambda m, f: (m, 0)),
            pl.BlockSpec((1, d), lambda m, f: (0, 0)),
            pl.BlockSpec((1, d), lambda m, f: (0, 0)),
            pl.BlockSpec((1, d), lambda m, f: (0, 0)),
            pl.BlockSpec((_TF, d), lambda m, f: (f, 0)),
            pl.BlockSpec((1, _TF), lambda m, f: (0, f)),
            pl.BlockSpec((d, _TF), lambda m, f: (0, f)),
            pl.BlockSpec((1, d), lambda m, f: (0, 0)),
        ],
        out_specs=pl.BlockSpec((_TM, d), lambda m, f: (m, 0)),
        out_shape=jax.ShapeDtypeStruct((n, d), jnp.float32),
        scratch_shapes=[
            pltpu.VMEM((_TM, d), jnp.float8_e4m3fn),
            pltpu.VMEM((_TM, 1), jnp.float32),
        ],
        compiler_params=pltpu.CompilerParams(
            dimension_semantics=("parallel", "arbitrary")),
    )(x, c0, g, b, w1, b1, w2, b2)
    return out.reshape(orig_shape)
